# baseline (speedup 1.0000x reference)
import functools

import jax
import jax.numpy as jnp
from jax import lax
from jax.experimental import pallas as pl
from jax.experimental.pallas import tpu as pltpu
from jax.experimental.pallas import tpu_sc as plsc

E = 320000
D = 128
DW = D // 2
B = 256
NW = 32
NCHUNK = E // B
L = 16


def _body(idx_hbm, z_hbm, w_hbm, out_hbm,
          ib0, srow0, drow0, wrow0,
          ib1, srow1, drow1, wrow1, outb,
          sem_s0, sem_d0, sem_w0, sem_s1, sem_d1, sem_w1):
  wid = lax.axis_index("s") * 2 + lax.axis_index("c")
  nmine = (NCHUNK - wid + NW - 1) // NW
  lane = lax.iota(jnp.int32, L)
  sixteen = jnp.full((L,), 16, jnp.int32)
  hi_mask = jnp.full((L,), -65536, jnp.int32)

  sets = ((ib0, srow0, drow0, wrow0, sem_s0, sem_d0, sem_w0),
          (ib1, srow1, drow1, wrow1, sem_s1, sem_d1, sem_w1))

  def fire(k, st):
    ib, sr, dr, wr, ss, sd, sw = st
    base3 = (wid + k * NW) * (3 * B)
    pltpu.sync_copy(idx_hbm.at[pl.ds(base3, 3 * B)], ib)
    pltpu.async_copy(z_hbm.at[ib.at[pl.ds(0, B)]], sr, ss)
    pltpu.async_copy(z_hbm.at[ib.at[pl.ds(B, B)]], dr, sd)
    pltpu.async_copy(w_hbm.at[ib.at[pl.ds(2 * B, B)]], wr, sw)

  def consume(k, st):
    ib, sr, dr, wr, ss, sd, sw = st
    base = (wid + k * NW) * B
    pltpu.make_async_copy(z_hbm.at[ib.at[pl.ds(0, B)]], sr, ss).wait()
    pltpu.make_async_copy(z_hbm.at[ib.at[pl.ds(B, B)]], dr, sd).wait()
    pltpu.make_async_copy(w_hbm.at[ib.at[pl.ds(2 * B, B)]], wr, sw).wait()

    @plsc.parallel_loop(0, B // L, 1, unroll=1)
    def _grp_loop(g):
      accs = []
      for i in range(L):
        e = g * L + i
        acc = None
        for j in range(DW // L):
          sb = plsc.bitcast(sr[e, pl.ds(j * L, L)], jnp.bfloat16)
          db = plsc.bitcast(dr[e, pl.ds(j * L, L)], jnp.bfloat16)
          wb = plsc.bitcast(wr[e, pl.ds(j * L, L)], jnp.bfloat16)
          p = plsc.bitcast(sb * db * wb, jnp.int32)
          t = (plsc.bitcast(p << sixteen, jnp.float32)
               + plsc.bitcast(p & hi_mask, jnp.float32))
          acc = t if acc is None else acc + t
        accs.append(acc)
      vecs = accs
      for sh in (1, 2, 4, 8):
        m = (lane & sh) == 0
        idx = jnp.bitwise_xor(lane, sh)
        nxt = []
        for q in range(0, len(vecs), 2):
          a, b = vecs[q], vecs[q + 1]
          t2 = jnp.where(m, b, a)
          nxt.append(jnp.where(m, a, b)
                     + t2.at[idx].get(mode="promise_in_bounds"))
        vecs = nxt
      vals = 1.0 / (1.0 + jnp.exp(-vecs[0]))
      outb[pl.ds(g * L, L)] = vals

    pltpu.sync_copy(outb, out_hbm.at[pl.ds(base, B)])

  fire(0, sets[0])

  def chunk_body(k, _):
    even = k % 2 == 0

    @pl.when((k + 1 < nmine) & even)
    def _():
      fire(k + 1, sets[1])

    @pl.when((k + 1 < nmine) & jnp.logical_not(even))
    def _():
      fire(k + 1, sets[0])

    @pl.when(even)
    def _():
      consume(k, sets[0])

    @pl.when(jnp.logical_not(even))
    def _():
      consume(k, sets[1])

    return 0

  lax.fori_loop(0, nmine, chunk_body, 0)


@functools.partial(jax.jit, donate_argnums=())
def _decode(idx3, z, w):
  mesh = plsc.VectorSubcoreMesh(core_axis_name="c", subcore_axis_name="s")
  f = pl.kernel(
      _body,
      out_type=jax.ShapeDtypeStruct((E,), jnp.float32),
      mesh=mesh,
      compiler_params=pltpu.CompilerParams(needs_layout_passes=False,
                                           use_tc_tiling_on_sc=False),
      scratch_types=[
          pltpu.VMEM((3 * B,), jnp.int32),
          pltpu.VMEM((B, DW), jnp.int32),
          pltpu.VMEM((B, DW), jnp.int32),
          pltpu.VMEM((B, DW), jnp.int32),
          pltpu.VMEM((3 * B,), jnp.int32),
          pltpu.VMEM((B, DW), jnp.int32),
          pltpu.VMEM((B, DW), jnp.int32),
          pltpu.VMEM((B, DW), jnp.int32),
          pltpu.VMEM((B,), jnp.float32),
          pltpu.SemaphoreType.DMA,
          pltpu.SemaphoreType.DMA,
          pltpu.SemaphoreType.DMA,
          pltpu.SemaphoreType.DMA,
          pltpu.SemaphoreType.DMA,
          pltpu.SemaphoreType.DMA,
      ],
  )
  return f(idx3, z, w)


def kernel(z, edge_index, edge_type, weight):
  src = edge_index[0].astype(jnp.int32)
  dst = edge_index[1].astype(jnp.int32)
  et = edge_type.astype(jnp.int32)
  idx3 = jnp.stack([src.reshape(NCHUNK, B), dst.reshape(NCHUNK, B),
                    et.reshape(NCHUNK, B)], axis=1).reshape(NCHUNK * 3 * B)
  z_i = lax.bitcast_convert_type(
      z.astype(jnp.bfloat16).reshape(z.shape[0], DW, 2), jnp.int32)
  w_i = lax.bitcast_convert_type(
      weight.astype(jnp.bfloat16).reshape(weight.shape[0], DW, 2), jnp.int32)
  return _decode(idx3, z_i, w_i)


# speedup vs baseline: 1.0236x; 1.0236x over previous
import functools

import jax
import jax.numpy as jnp
from jax import lax
from jax.experimental import pallas as pl
from jax.experimental.pallas import tpu as pltpu
from jax.experimental.pallas import tpu_sc as plsc

E = 320000
D = 128
DW = D // 2
B = 320
NW = 32
NCHUNK = E // B
L = 16


def _body(idx_hbm, z_hbm, w_hbm, out_hbm,
          ib0, srow0, drow0, wrow0,
          ib1, srow1, drow1, wrow1, outb,
          sem_s0, sem_d0, sem_w0, sem_s1, sem_d1, sem_w1):
  wid = lax.axis_index("s") * 2 + lax.axis_index("c")
  nmine = (NCHUNK - wid + NW - 1) // NW
  lane = lax.iota(jnp.int32, L)
  sixteen = jnp.full((L,), 16, jnp.int32)
  hi_mask = jnp.full((L,), -65536, jnp.int32)

  sets = ((ib0, srow0, drow0, wrow0, sem_s0, sem_d0, sem_w0),
          (ib1, srow1, drow1, wrow1, sem_s1, sem_d1, sem_w1))

  def fire(k, st):
    ib, sr, dr, wr, ss, sd, sw = st
    base3 = (wid + k * NW) * (3 * B)
    pltpu.sync_copy(idx_hbm.at[pl.ds(base3, 3 * B)], ib)
    pltpu.async_copy(z_hbm.at[ib.at[pl.ds(0, B)]], sr, ss)
    pltpu.async_copy(z_hbm.at[ib.at[pl.ds(B, B)]], dr, sd)
    pltpu.async_copy(w_hbm.at[ib.at[pl.ds(2 * B, B)]], wr, sw)

  def consume(k, st):
    ib, sr, dr, wr, ss, sd, sw = st
    base = (wid + k * NW) * B
    pltpu.make_async_copy(z_hbm.at[ib.at[pl.ds(0, B)]], sr, ss).wait()
    pltpu.make_async_copy(z_hbm.at[ib.at[pl.ds(B, B)]], dr, sd).wait()
    pltpu.make_async_copy(w_hbm.at[ib.at[pl.ds(2 * B, B)]], wr, sw).wait()

    lane0 = lane == 0

    @plsc.parallel_loop(0, B, 1, unroll=4)
    def _edge_loop(e):
        acc = None
        for j in range(DW // L):
          sb = plsc.bitcast(sr[e, pl.ds(j * L, L)], jnp.bfloat16)
          db = plsc.bitcast(dr[e, pl.ds(j * L, L)], jnp.bfloat16)
          wb = plsc.bitcast(wr[e, pl.ds(j * L, L)], jnp.bfloat16)
          p = plsc.bitcast(sb * db * wb, jnp.int32)
          t = (plsc.bitcast(p << sixteen, jnp.float32)
               + plsc.bitcast(p & hi_mask, jnp.float32))
          acc = t if acc is None else acc + t
        v = acc
        for sh in (8, 4, 2, 1):
          v = v + v.at[jnp.bitwise_xor(lane, sh)].get(mode="promise_in_bounds")
        plsc.store_scatter(outb, [jnp.full((L,), e, jnp.int32)], v,
                           mask=lane0)

    def grp_body(g, _):
      vals = outb[pl.ds(g * L, L)]
      outb[pl.ds(g * L, L)] = 1.0 / (1.0 + jnp.exp(-vals))
      return 0

    lax.fori_loop(0, B // L, grp_body, 0)
    pltpu.sync_copy(outb, out_hbm.at[pl.ds(base, B)])

  fire(0, sets[0])

  def chunk_body(k, _):
    even = k % 2 == 0

    @pl.when((k + 1 < nmine) & even)
    def _():
      fire(k + 1, sets[1])

    @pl.when((k + 1 < nmine) & jnp.logical_not(even))
    def _():
      fire(k + 1, sets[0])

    @pl.when(even)
    def _():
      consume(k, sets[0])

    @pl.when(jnp.logical_not(even))
    def _():
      consume(k, sets[1])

    return 0

  lax.fori_loop(0, nmine, chunk_body, 0)


@functools.partial(jax.jit, donate_argnums=())
def _decode(idx3, z, w):
  mesh = plsc.VectorSubcoreMesh(core_axis_name="c", subcore_axis_name="s")
  f = pl.kernel(
      _body,
      out_type=jax.ShapeDtypeStruct((E,), jnp.float32),
      mesh=mesh,
      compiler_params=pltpu.CompilerParams(needs_layout_passes=False,
                                           use_tc_tiling_on_sc=False),
      scratch_types=[
          pltpu.VMEM((3 * B,), jnp.int32),
          pltpu.VMEM((B, DW), jnp.int32),
          pltpu.VMEM((B, DW), jnp.int32),
          pltpu.VMEM((B, DW), jnp.int32),
          pltpu.VMEM((3 * B,), jnp.int32),
          pltpu.VMEM((B, DW), jnp.int32),
          pltpu.VMEM((B, DW), jnp.int32),
          pltpu.VMEM((B, DW), jnp.int32),
          pltpu.VMEM((B,), jnp.float32),
          pltpu.SemaphoreType.DMA,
          pltpu.SemaphoreType.DMA,
          pltpu.SemaphoreType.DMA,
          pltpu.SemaphoreType.DMA,
          pltpu.SemaphoreType.DMA,
          pltpu.SemaphoreType.DMA,
      ],
  )
  return f(idx3, z, w)


def kernel(z, edge_index, edge_type, weight):
  src = edge_index[0].astype(jnp.int32)
  dst = edge_index[1].astype(jnp.int32)
  et = edge_type.astype(jnp.int32)
  idx3 = jnp.stack([src.reshape(NCHUNK, B), dst.reshape(NCHUNK, B),
                    et.reshape(NCHUNK, B)], axis=1).reshape(NCHUNK * 3 * B)
  z_i = lax.bitcast_convert_type(
      z.astype(jnp.bfloat16).reshape(z.shape[0], DW, 2), jnp.int32)
  w_i = lax.bitcast_convert_type(
      weight.astype(jnp.bfloat16).reshape(weight.shape[0], DW, 2), jnp.int32)
  return _decode(idx3, z_i, w_i)


# speedup vs baseline: 1.0656x; 1.0410x over previous
import functools

import jax
import jax.numpy as jnp
from jax import lax
from jax.experimental import pallas as pl
from jax.experimental.pallas import tpu as pltpu
from jax.experimental.pallas import tpu_sc as plsc

E = 320000
D = 128
DW = D // 2
B = 256
NW = 32
NCHUNK = E // B
L = 16


def _body(idx_hbm, z_hbm, w_hbm, out_hbm,
          ib0, srow0, drow0, wrow0,
          ib1, srow1, drow1, wrow1, ib2, outb,
          sem_s0, sem_d0, sem_w0, sem_s1, sem_d1, sem_w1,
          sem_i0, sem_i1, sem_i2):
  wid = lax.axis_index("s") * 2 + lax.axis_index("c")
  nmine = (NCHUNK - wid + NW - 1) // NW
  lane = lax.iota(jnp.int32, L)
  sixteen = jnp.full((L,), 16, jnp.int32)
  hi_mask = jnp.full((L,), -65536, jnp.int32)

  ibs = (ib0, ib1, ib2)
  isems = (sem_i0, sem_i1, sem_i2)
  rowsets = ((srow0, drow0, wrow0, sem_s0, sem_d0, sem_w0),
             (srow1, drow1, wrow1, sem_s1, sem_d1, sem_w1))

  def fire_idx(k, p3):
    base3 = (wid + k * NW) * (3 * B)
    pltpu.async_copy(idx_hbm.at[pl.ds(base3, 3 * B)], ibs[p3], isems[p3])

  def fire_gath(k, p3, p2):
    ib = ibs[p3]
    base3 = (wid + k * NW) * (3 * B)
    pltpu.make_async_copy(idx_hbm.at[pl.ds(base3, 3 * B)], ib,
                          isems[p3]).wait()
    sr, dr, wr, ss, sd, sw = rowsets[p2]
    pltpu.async_copy(z_hbm.at[ib.at[pl.ds(0, B)]], sr, ss)
    pltpu.async_copy(z_hbm.at[ib.at[pl.ds(B, B)]], dr, sd)
    pltpu.async_copy(w_hbm.at[ib.at[pl.ds(2 * B, B)]], wr, sw)

  def consume(k, p3, p2):
    ib = ibs[p3]
    sr, dr, wr, ss, sd, sw = rowsets[p2]
    base = (wid + k * NW) * B
    pltpu.make_async_copy(z_hbm.at[ib.at[pl.ds(0, B)]], sr, ss).wait()
    pltpu.make_async_copy(z_hbm.at[ib.at[pl.ds(B, B)]], dr, sd).wait()
    pltpu.make_async_copy(w_hbm.at[ib.at[pl.ds(2 * B, B)]], wr, sw).wait()

    lane0 = lane == 0

    @plsc.parallel_loop(0, B, 1, unroll=4)
    def _edge_loop(e):
        acc = None
        for j in range(DW // L):
          sb = plsc.bitcast(sr[e, pl.ds(j * L, L)], jnp.bfloat16)
          db = plsc.bitcast(dr[e, pl.ds(j * L, L)], jnp.bfloat16)
          wb = plsc.bitcast(wr[e, pl.ds(j * L, L)], jnp.bfloat16)
          p = plsc.bitcast(sb * db * wb, jnp.int32)
          t = (plsc.bitcast(p << sixteen, jnp.float32)
               + plsc.bitcast(p & hi_mask, jnp.float32))
          acc = t if acc is None else acc + t
        v = acc
        for sh in (8, 4, 2, 1):
          v = v + v.at[jnp.bitwise_xor(lane, sh)].get(mode="promise_in_bounds")
        plsc.store_scatter(outb, [jnp.full((L,), e, jnp.int32)], v,
                           mask=lane0)

    def grp_body(g, _):
      vals = outb[pl.ds(g * L, L)]
      outb[pl.ds(g * L, L)] = 1.0 / (1.0 + jnp.exp(-vals))
      return 0

    lax.fori_loop(0, B // L, grp_body, 0)
    pltpu.sync_copy(outb, out_hbm.at[pl.ds(base, B)])

  fire_idx(0, 0)
  fire_idx(1, 1)
  fire_gath(0, 0, 0)

  def chunk_body(k, _):
    for p3 in range(3):
      @pl.when((k + 2 < nmine) & ((k + 2) % 3 == p3))
      def _(p3=p3):
        fire_idx(k + 2, p3)

    for p3 in range(3):
      for p2 in range(2):
        @pl.when((k + 1 < nmine) & ((k + 1) % 3 == p3)
                 & ((k + 1) % 2 == p2))
        def _(p3=p3, p2=p2):
          fire_gath(k + 1, p3, p2)

    for p3 in range(3):
      for p2 in range(2):
        @pl.when((k % 3 == p3) & (k % 2 == p2))
        def _(p3=p3, p2=p2):
          consume(k, p3, p2)

    return 0

  lax.fori_loop(0, nmine, chunk_body, 0)


@functools.partial(jax.jit, donate_argnums=())
def _decode(idx3, z, w):
  mesh = plsc.VectorSubcoreMesh(core_axis_name="c", subcore_axis_name="s")
  f = pl.kernel(
      _body,
      out_type=jax.ShapeDtypeStruct((E,), jnp.float32),
      mesh=mesh,
      compiler_params=pltpu.CompilerParams(needs_layout_passes=False,
                                           use_tc_tiling_on_sc=False),
      scratch_types=[
          pltpu.VMEM((3 * B,), jnp.int32),
          pltpu.VMEM((B, DW), jnp.int32),
          pltpu.VMEM((B, DW), jnp.int32),
          pltpu.VMEM((B, DW), jnp.int32),
          pltpu.VMEM((3 * B,), jnp.int32),
          pltpu.VMEM((B, DW), jnp.int32),
          pltpu.VMEM((B, DW), jnp.int32),
          pltpu.VMEM((B, DW), jnp.int32),
          pltpu.VMEM((3 * B,), jnp.int32),
          pltpu.VMEM((B,), jnp.float32),
          pltpu.SemaphoreType.DMA,
          pltpu.SemaphoreType.DMA,
          pltpu.SemaphoreType.DMA,
          pltpu.SemaphoreType.DMA,
          pltpu.SemaphoreType.DMA,
          pltpu.SemaphoreType.DMA,
          pltpu.SemaphoreType.DMA,
          pltpu.SemaphoreType.DMA,
          pltpu.SemaphoreType.DMA,
      ],
  )
  return f(idx3, z, w)


def kernel(z, edge_index, edge_type, weight):
  src = edge_index[0].astype(jnp.int32)
  dst = edge_index[1].astype(jnp.int32)
  et = edge_type.astype(jnp.int32)
  idx3 = jnp.stack([src.reshape(NCHUNK, B), dst.reshape(NCHUNK, B),
                    et.reshape(NCHUNK, B)], axis=1).reshape(NCHUNK * 3 * B)
  z_i = lax.bitcast_convert_type(
      z.astype(jnp.bfloat16).reshape(z.shape[0], DW, 2), jnp.int32)
  w_i = lax.bitcast_convert_type(
      weight.astype(jnp.bfloat16).reshape(weight.shape[0], DW, 2), jnp.int32)
  return _decode(idx3, z_i, w_i)


# speedup vs baseline: 1.1589x; 1.0876x over previous
import functools

import jax
import jax.numpy as jnp
from jax import lax
from jax.experimental import pallas as pl
from jax.experimental.pallas import tpu as pltpu
from jax.experimental.pallas import tpu_sc as plsc

E = 320000
D = 128
DW = D // 2
B = 256
NW = 32
NCHUNK = E // B
L = 16


def _body(src_hbm, dst_hbm, et_hbm, z_hbm, w_hbm, out_hbm,
          ib0, srow0, drow0, wrow0,
          ib1, srow1, drow1, wrow1, ib2, outb,
          sem_s0, sem_d0, sem_w0, sem_s1, sem_d1, sem_w1,
          sem_i0, sem_i1, sem_i2):
  wid = lax.axis_index("s") * 2 + lax.axis_index("c")
  nmine = (NCHUNK - wid + NW - 1) // NW
  lane = lax.iota(jnp.int32, L)
  sixteen = jnp.full((L,), 16, jnp.int32)
  hi_mask = jnp.full((L,), -65536, jnp.int32)

  ibs = (ib0, ib1, ib2)
  isems = (sem_i0, sem_i1, sem_i2)
  rowsets = ((srow0, drow0, wrow0, sem_s0, sem_d0, sem_w0),
             (srow1, drow1, wrow1, sem_s1, sem_d1, sem_w1))

  def fire_idx(k, p3):
    base = (wid + k * NW) * B
    ib = ibs[p3]
    pltpu.async_copy(src_hbm.at[pl.ds(base, B)], ib.at[pl.ds(0, B)],
                     isems[p3])
    pltpu.async_copy(dst_hbm.at[pl.ds(base, B)], ib.at[pl.ds(B, B)],
                     isems[p3])
    pltpu.async_copy(et_hbm.at[pl.ds(base, B)], ib.at[pl.ds(2 * B, B)],
                     isems[p3])

  def fire_gath(k, p3, p2):
    ib = ibs[p3]
    base = (wid + k * NW) * B
    pltpu.make_async_copy(src_hbm.at[pl.ds(base, B)], ib.at[pl.ds(0, B)],
                          isems[p3]).wait()
    pltpu.make_async_copy(dst_hbm.at[pl.ds(base, B)], ib.at[pl.ds(B, B)],
                          isems[p3]).wait()
    pltpu.make_async_copy(et_hbm.at[pl.ds(base, B)], ib.at[pl.ds(2 * B, B)],
                          isems[p3]).wait()
    sr, dr, wr, ss, sd, sw = rowsets[p2]
    pltpu.async_copy(z_hbm.at[ib.at[pl.ds(0, B)]], sr, ss)
    pltpu.async_copy(z_hbm.at[ib.at[pl.ds(B, B)]], dr, sd)
    pltpu.async_copy(w_hbm.at[ib.at[pl.ds(2 * B, B)]], wr, sw)

  def consume(k, p3, p2):
    ib = ibs[p3]
    sr, dr, wr, ss, sd, sw = rowsets[p2]
    base = (wid + k * NW) * B
    pltpu.make_async_copy(z_hbm.at[ib.at[pl.ds(0, B)]], sr, ss).wait()
    pltpu.make_async_copy(z_hbm.at[ib.at[pl.ds(B, B)]], dr, sd).wait()
    pltpu.make_async_copy(w_hbm.at[ib.at[pl.ds(2 * B, B)]], wr, sw).wait()

    lane0 = lane == 0

    @plsc.parallel_loop(0, B, 1, unroll=4)
    def _edge_loop(e):
        acc = None
        for j in range(DW // L):
          sb = plsc.bitcast(sr[e, pl.ds(j * L, L)], jnp.bfloat16)
          db = plsc.bitcast(dr[e, pl.ds(j * L, L)], jnp.bfloat16)
          wb = plsc.bitcast(wr[e, pl.ds(j * L, L)], jnp.bfloat16)
          p = plsc.bitcast(sb * db * wb, jnp.int32)
          t = (plsc.bitcast(p << sixteen, jnp.float32)
               + plsc.bitcast(p & hi_mask, jnp.float32))
          acc = t if acc is None else acc + t
        v = acc
        for sh in (8, 4, 2, 1):
          v = v + v.at[jnp.bitwise_xor(lane, sh)].get(mode="promise_in_bounds")
        plsc.store_scatter(outb, [jnp.full((L,), e, jnp.int32)], v,
                           mask=lane0)

    def grp_body(g, _):
      vals = outb[pl.ds(g * L, L)]
      outb[pl.ds(g * L, L)] = 1.0 / (1.0 + jnp.exp(-vals))
      return 0

    lax.fori_loop(0, B // L, grp_body, 0)
    pltpu.sync_copy(outb, out_hbm.at[pl.ds(base, B)])

  fire_idx(0, 0)
  fire_idx(1, 1)
  fire_gath(0, 0, 0)

  def chunk_body(k, _):
    for p3 in range(3):
      @pl.when((k + 2 < nmine) & ((k + 2) % 3 == p3))
      def _(p3=p3):
        fire_idx(k + 2, p3)

    for p3 in range(3):
      for p2 in range(2):
        @pl.when((k + 1 < nmine) & ((k + 1) % 3 == p3)
                 & ((k + 1) % 2 == p2))
        def _(p3=p3, p2=p2):
          fire_gath(k + 1, p3, p2)

    for p3 in range(3):
      for p2 in range(2):
        @pl.when((k % 3 == p3) & (k % 2 == p2))
        def _(p3=p3, p2=p2):
          consume(k, p3, p2)

    return 0

  lax.fori_loop(0, nmine, chunk_body, 0)


@functools.partial(jax.jit, donate_argnums=())
def _decode(src, dst, et, z, w):
  mesh = plsc.VectorSubcoreMesh(core_axis_name="c", subcore_axis_name="s")
  f = pl.kernel(
      _body,
      out_type=jax.ShapeDtypeStruct((E,), jnp.float32),
      mesh=mesh,
      compiler_params=pltpu.CompilerParams(needs_layout_passes=False,
                                           use_tc_tiling_on_sc=False),
      scratch_types=[
          pltpu.VMEM((3 * B,), jnp.int32),
          pltpu.VMEM((B, DW), jnp.int32),
          pltpu.VMEM((B, DW), jnp.int32),
          pltpu.VMEM((B, DW), jnp.int32),
          pltpu.VMEM((3 * B,), jnp.int32),
          pltpu.VMEM((B, DW), jnp.int32),
          pltpu.VMEM((B, DW), jnp.int32),
          pltpu.VMEM((B, DW), jnp.int32),
          pltpu.VMEM((3 * B,), jnp.int32),
          pltpu.VMEM((B,), jnp.float32),
          pltpu.SemaphoreType.DMA,
          pltpu.SemaphoreType.DMA,
          pltpu.SemaphoreType.DMA,
          pltpu.SemaphoreType.DMA,
          pltpu.SemaphoreType.DMA,
          pltpu.SemaphoreType.DMA,
          pltpu.SemaphoreType.DMA,
          pltpu.SemaphoreType.DMA,
          pltpu.SemaphoreType.DMA,
      ],
  )
  return f(src, dst, et, z, w)


def kernel(z, edge_index, edge_type, weight):
  src = edge_index[0].astype(jnp.int32)
  dst = edge_index[1].astype(jnp.int32)
  et = edge_type.astype(jnp.int32)
  z_i = lax.bitcast_convert_type(
      z.astype(jnp.bfloat16).reshape(z.shape[0], DW, 2), jnp.int32)
  w_i = lax.bitcast_convert_type(
      weight.astype(jnp.bfloat16).reshape(weight.shape[0], DW, 2), jnp.int32)
  return _decode(src, dst, et, z_i, w_i)
